# baseline (speedup 1.0000x reference)
import functools

import jax
import jax.numpy as jnp
import numpy as np
from jax import lax
from jax.experimental import pallas as pl
from jax.experimental.pallas import tpu as pltpu
from jax.experimental.pallas import tpu_sc as plsc

_B = 1024
_T = 200
_P = 196
_NO = 26
_D = 128
_NUM_REMAIN = _NO // 2
_BB = 32

_NW = 32
_SW = _B // _NW
_EW = _SW * _NUM_REMAIN
_CH = 104
_NCH = _EW // _CH


def _threefry2x32(k1, k2, x1, x2):
    ks = [np.uint32(k1), np.uint32(k2),
          np.uint32(k1) ^ np.uint32(k2) ^ np.uint32(0x1BD11BDA)]
    rotations = [(13, 15, 26, 6), (17, 29, 16, 24)]
    x = [x1 + ks[0], x2 + ks[1]]
    for i in range(5):
        for r in rotations[i % 2]:
            x[0] = x[0] + x[1]
            x[1] = (x[1] << np.uint32(r)) | (x[1] >> np.uint32(32 - r))
            x[1] = x[1] ^ x[0]
        x[0] = x[0] + ks[(i + 1) % 3]
        x[1] = x[1] + ks[(i + 2) % 3] + np.uint32(i + 1)
    return x


def _uniform_f32(seed, shape):
    size = int(np.prod(shape))
    k1 = np.uint32(np.uint64(seed) >> np.uint64(32))
    k2 = np.uint32(np.uint64(seed) & np.uint64(0xFFFFFFFF))
    hi = np.zeros(size, dtype=np.uint32)
    lo = np.arange(size, dtype=np.uint32)
    with np.errstate(over="ignore"):
        r = _threefry2x32(k1, k2, hi, lo)
    bits = r[0] ^ r[1]
    fb = (bits >> np.uint32(9)) | np.uint32(0x3F800000)
    return (fb.view(np.float32) - np.float32(1.0)).reshape(shape)


@functools.lru_cache(maxsize=1)
def _index_constants():
    noise = _uniform_f32(42, (_B, _NO))
    shuffle = np.argsort(noise, axis=-1, kind="stable").astype(np.int32)
    remain = shuffle[:, :_NUM_REMAIN]
    masked = shuffle[:, _NUM_REMAIN:]
    revert = np.argsort(shuffle, axis=-1, kind="stable").astype(np.int32)
    return remain, masked, revert


@functools.lru_cache(maxsize=1)
def _sc_gather_constants():
    remain, _, _ = _index_constants()
    b = np.repeat(np.arange(_B, dtype=np.int32), _NUM_REMAIN)
    j = np.tile(np.arange(_NUM_REMAIN, dtype=np.int32), _B)
    r = remain.reshape(-1)
    src_o = (b * _NO + r).reshape(_NW * _NCH, _CH)
    src_pe = (3 + r).reshape(_NW * _NCH, _CH)
    dst_o = (b * (1 + _NUM_REMAIN) + 1 + j).reshape(_NW * _NCH, _CH)
    dst_g = (np.arange(_B, dtype=np.int32) * (1 + _NUM_REMAIN)).reshape(_NW, _SW)
    return src_o, src_pe, dst_o, dst_g


def _tc_body(t_ref, i_ref, pe_ref, to_ref, io_ref):
    to_ref[...] = t_ref[...] + pe_ref[1:2, :][None]
    io_ref[...] = i_ref[...] + pe_ref[2:3, :][None]


def _sc_body(others_ref, pe_ref, gt_ref, src_o_ref, src_pe_ref, dst_o_ref,
             dst_g_ref, out_ref,
             idx_o, idx_pe, idx_d, idx_g, rows, perows, gtbuf, gtrow, pe0row,
             gsem, ssem):
    w = lax.axis_index("s") * 2 + lax.axis_index("c")
    gathers = []
    for c in range(_NCH):
        pltpu.sync_copy(src_o_ref.at[w * _NCH + c], idx_o.at[c])
        pltpu.sync_copy(src_pe_ref.at[w * _NCH + c], idx_pe.at[c])
        gathers.append(pltpu.async_copy(
            others_ref.at[idx_o.at[c]], rows.at[c], gsem))
        gathers.append(pltpu.async_copy(
            pe_ref.at[idx_pe.at[c]], perows.at[c], gsem))
    pltpu.sync_copy(gt_ref, gtrow)
    pltpu.sync_copy(pe_ref.at[pl.ds(0, 1)], pe0row)

    def fill_gt(i, _):
        for v in range(8):
            gtbuf[i, pl.ds(v * 16, 16)] = (
                gtrow[0, pl.ds(v * 16, 16)] + pe0row[0, pl.ds(v * 16, 16)])
        return 0

    lax.fori_loop(0, _SW, fill_gt, 0)
    pltpu.sync_copy(dst_g_ref.at[w], idx_g)
    for g in gathers:
        g.wait()
    scatters = []
    for c in range(_NCH):
        def add_pe(k, _, c=c):
            for v in range(8):
                rows[c, k, pl.ds(v * 16, 16)] = (
                    rows[c, k, pl.ds(v * 16, 16)]
                    + perows[c, k, pl.ds(v * 16, 16)])
            return 0

        lax.fori_loop(0, _CH, add_pe, 0)
        pltpu.sync_copy(dst_o_ref.at[w * _NCH + c], idx_d.at[c])
        scatters.append(pltpu.async_copy(
            rows.at[c], out_ref.at[idx_d.at[c]], ssem))
    scatters.append(pltpu.async_copy(gtbuf, out_ref.at[idx_g], ssem))
    for s in scatters:
        s.wait()


@jax.jit
def kernel(temporal_x, img_x, others_x, pos_emb, global_token):
    remain, masked, revert = _index_constants()
    src_o, src_pe, dst_o, dst_g = _sc_gather_constants()

    mesh = plsc.VectorSubcoreMesh(core_axis_name="c", subcore_axis_name="s")
    sc_call = functools.partial(
        pl.kernel, _sc_body, mesh=mesh,
        out_type=jax.ShapeDtypeStruct((_B * (1 + _NUM_REMAIN), _D), jnp.float32),
        scratch_types=[
            pltpu.VMEM((_NCH, _CH), jnp.int32),
            pltpu.VMEM((_NCH, _CH), jnp.int32),
            pltpu.VMEM((_NCH, _CH), jnp.int32),
            pltpu.VMEM((_SW,), jnp.int32),
            pltpu.VMEM((_NCH, _CH, _D), jnp.float32),
            pltpu.VMEM((_NCH, _CH, _D), jnp.float32),
            pltpu.VMEM((_SW, _D), jnp.float32),
            pltpu.VMEM((1, _D), jnp.float32),
            pltpu.VMEM((1, _D), jnp.float32),
            pltpu.SemaphoreType.DMA,
            pltpu.SemaphoreType.DMA,
        ],
    )
    or_flat = sc_call()(
        others_x.reshape(_B * _NO, _D),
        pos_emb,
        global_token,
        jnp.asarray(src_o),
        jnp.asarray(src_pe),
        jnp.asarray(dst_o),
        jnp.asarray(dst_g),
    )
    or_out = or_flat.reshape(_B, 1 + _NUM_REMAIN, _D)

    grid = (_B // _BB,)
    t_out, i_out = pl.pallas_call(
        _tc_body,
        grid=grid,
        in_specs=[
            pl.BlockSpec((_BB, _T, _D), lambda i: (i, 0, 0)),
            pl.BlockSpec((_BB, _P, _D), lambda i: (i, 0, 0)),
            pl.BlockSpec((pos_emb.shape[0], _D), lambda i: (0, 0)),
        ],
        out_specs=[
            pl.BlockSpec((_BB, _T, _D), lambda i: (i, 0, 0)),
            pl.BlockSpec((_BB, _P, _D), lambda i: (i, 0, 0)),
        ],
        out_shape=[
            jax.ShapeDtypeStruct((_B, _T, _D), jnp.float32),
            jax.ShapeDtypeStruct((_B, _P, _D), jnp.float32),
        ],
    )(temporal_x, img_x, pos_emb)
    return (t_out, i_out, or_out,
            jnp.asarray(remain), jnp.asarray(masked), jnp.asarray(revert))


# speedup vs baseline: 1.0097x; 1.0097x over previous
import functools

import jax
import jax.numpy as jnp
import numpy as np
from jax import lax
from jax.experimental import pallas as pl
from jax.experimental.pallas import tpu as pltpu
from jax.experimental.pallas import tpu_sc as plsc

_B = 1024
_T = 200
_P = 196
_NO = 26
_D = 128
_NUM_REMAIN = _NO // 2
_OPAD = 16
_BB = 32
_BA = 16

_NW = 32
_SW = _B // _NW
_EW = _SW * _NUM_REMAIN
_CH = 104
_NCH = _EW // _CH


def _threefry2x32(k1, k2, x1, x2):
    ks = [np.uint32(k1), np.uint32(k2),
          np.uint32(k1) ^ np.uint32(k2) ^ np.uint32(0x1BD11BDA)]
    rotations = [(13, 15, 26, 6), (17, 29, 16, 24)]
    x = [x1 + ks[0], x2 + ks[1]]
    for i in range(5):
        for r in rotations[i % 2]:
            x[0] = x[0] + x[1]
            x[1] = (x[1] << np.uint32(r)) | (x[1] >> np.uint32(32 - r))
            x[1] = x[1] ^ x[0]
        x[0] = x[0] + ks[(i + 1) % 3]
        x[1] = x[1] + ks[(i + 2) % 3] + np.uint32(i + 1)
    return x


def _uniform_f32(seed, shape):
    size = int(np.prod(shape))
    k1 = np.uint32(np.uint64(seed) >> np.uint64(32))
    k2 = np.uint32(np.uint64(seed) & np.uint64(0xFFFFFFFF))
    hi = np.zeros(size, dtype=np.uint32)
    lo = np.arange(size, dtype=np.uint32)
    with np.errstate(over="ignore"):
        r = _threefry2x32(k1, k2, hi, lo)
    bits = r[0] ^ r[1]
    fb = (bits >> np.uint32(9)) | np.uint32(0x3F800000)
    return (fb.view(np.float32) - np.float32(1.0)).reshape(shape)


@functools.lru_cache(maxsize=1)
def _index_constants():
    noise = _uniform_f32(42, (_B, _NO))
    shuffle = np.argsort(noise, axis=-1, kind="stable").astype(np.int32)
    remain = shuffle[:, :_NUM_REMAIN]
    masked = shuffle[:, _NUM_REMAIN:]
    revert = np.argsort(shuffle, axis=-1, kind="stable").astype(np.int32)
    return remain, masked, revert


@functools.lru_cache(maxsize=1)
def _sc_gather_constants():
    remain, _, _ = _index_constants()
    b = np.repeat(np.arange(_B, dtype=np.int32), _NUM_REMAIN)
    j = np.tile(np.arange(_NUM_REMAIN, dtype=np.int32), _B)
    r = remain.reshape(-1)
    src_o = (b * _NO + r).reshape(_NW * _NCH, _CH)
    dst_o = (b * _OPAD + 1 + j).reshape(_NW * _NCH, _CH)
    dst_g = (np.arange(_B, dtype=np.int32) * _OPAD).reshape(_NW, _SW)
    return src_o, dst_o, dst_g


def _tc_a_body(o_ref, pe_ref, opf_ref):
    for s in range(_BA):
        opf_ref[pl.ds(s * _NO, _NO), :] = (
            o_ref[s] + pe_ref[3:3 + _NO, :])


def _tc_b_body(t_ref, i_ref, pe_ref, to_ref, io_ref):
    to_ref[...] = t_ref[...] + pe_ref[1:2, :][None]
    io_ref[...] = i_ref[...] + pe_ref[2:3, :][None]


def _sc_body(opf_ref, pe_ref, gt_ref, src_o_ref, dst_o_ref, dst_g_ref,
             out_ref,
             idx_o, idx_d, idx_g, rows, gtbuf, gtrow, pe0row, gsem, ssem):
    w = lax.axis_index("s") * 2 + lax.axis_index("c")
    gathers = []
    for c in range(_NCH):
        pltpu.sync_copy(src_o_ref.at[w * _NCH + c], idx_o.at[c])
        gathers.append(pltpu.async_copy(
            opf_ref.at[idx_o.at[c]], rows.at[c], gsem))
    pltpu.sync_copy(gt_ref, gtrow)
    pltpu.sync_copy(pe_ref.at[pl.ds(0, 1)], pe0row)

    def fill_gt(i, _):
        for v in range(8):
            gtbuf[i, pl.ds(v * 16, 16)] = (
                gtrow[0, pl.ds(v * 16, 16)] + pe0row[0, pl.ds(v * 16, 16)])
        return 0

    lax.fori_loop(0, _SW, fill_gt, 0)
    pltpu.sync_copy(dst_g_ref.at[w], idx_g)
    scatters = [pltpu.async_copy(gtbuf, out_ref.at[idx_g], ssem)]
    for c, g in enumerate(gathers):
        g.wait()
        pltpu.sync_copy(dst_o_ref.at[w * _NCH + c], idx_d.at[c])
        scatters.append(pltpu.async_copy(
            rows.at[c], out_ref.at[idx_d.at[c]], ssem))
    for s in scatters:
        s.wait()


@jax.jit
def kernel(temporal_x, img_x, others_x, pos_emb, global_token):
    remain, masked, revert = _index_constants()
    src_o, dst_o, dst_g = _sc_gather_constants()

    opf = pl.pallas_call(
        _tc_a_body,
        grid=(_B // _BA,),
        in_specs=[
            pl.BlockSpec((_BA, _NO, _D), lambda i: (i, 0, 0)),
            pl.BlockSpec((pos_emb.shape[0], _D), lambda i: (0, 0)),
        ],
        out_specs=pl.BlockSpec((_BA * _NO, _D), lambda i: (i, 0)),
        out_shape=jax.ShapeDtypeStruct((_B * _NO, _D), jnp.float32),
    )(others_x, pos_emb)

    mesh = plsc.VectorSubcoreMesh(core_axis_name="c", subcore_axis_name="s")
    or16_flat = pl.kernel(
        _sc_body,
        out_type=jax.ShapeDtypeStruct((_B * _OPAD, _D), jnp.float32),
        mesh=mesh,
        scratch_types=[
            pltpu.VMEM((_NCH, _CH), jnp.int32),
            pltpu.VMEM((_NCH, _CH), jnp.int32),
            pltpu.VMEM((_SW,), jnp.int32),
            pltpu.VMEM((_NCH, _CH, _D), jnp.float32),
            pltpu.VMEM((_SW, _D), jnp.float32),
            pltpu.VMEM((1, _D), jnp.float32),
            pltpu.VMEM((1, _D), jnp.float32),
            pltpu.SemaphoreType.DMA,
            pltpu.SemaphoreType.DMA,
        ],
    )(
        opf,
        pos_emb,
        global_token,
        jnp.asarray(src_o),
        jnp.asarray(dst_o),
        jnp.asarray(dst_g),
    )
    or_out = or16_flat.reshape(_B, _OPAD, _D)[:, :1 + _NUM_REMAIN, :]

    t_out, i_out = pl.pallas_call(
        _tc_b_body,
        grid=(_B // _BB,),
        in_specs=[
            pl.BlockSpec((_BB, _T, _D), lambda i: (i, 0, 0)),
            pl.BlockSpec((_BB, _P, _D), lambda i: (i, 0, 0)),
            pl.BlockSpec((pos_emb.shape[0], _D), lambda i: (0, 0)),
        ],
        out_specs=[
            pl.BlockSpec((_BB, _T, _D), lambda i: (i, 0, 0)),
            pl.BlockSpec((_BB, _P, _D), lambda i: (i, 0, 0)),
        ],
        out_shape=[
            jax.ShapeDtypeStruct((_B, _T, _D), jnp.float32),
            jax.ShapeDtypeStruct((_B, _P, _D), jnp.float32),
        ],
    )(temporal_x, img_x, pos_emb)

    return (t_out, i_out, or_out,
            jnp.asarray(remain), jnp.asarray(masked), jnp.asarray(revert))


# speedup vs baseline: 2.0813x; 2.0612x over previous
import functools

import jax
import jax.numpy as jnp
import numpy as np
from jax import lax
from jax.experimental import pallas as pl
from jax.experimental.pallas import tpu as pltpu
from jax.experimental.pallas import tpu_sc as plsc

_B = 1024
_T = 200
_P = 196
_NO = 26
_D = 128
_NUM_REMAIN = _NO // 2
_GB = 32

_NW = 32
_SW = _B // _NW
_EW = _SW * _NUM_REMAIN
_CH = 104
_NCH = _EW // _CH


def _threefry2x32(k1, k2, x1, x2):
    ks = [np.uint32(k1), np.uint32(k2),
          np.uint32(k1) ^ np.uint32(k2) ^ np.uint32(0x1BD11BDA)]
    rotations = [(13, 15, 26, 6), (17, 29, 16, 24)]
    x = [x1 + ks[0], x2 + ks[1]]
    for i in range(5):
        for r in rotations[i % 2]:
            x[0] = x[0] + x[1]
            x[1] = (x[1] << np.uint32(r)) | (x[1] >> np.uint32(32 - r))
            x[1] = x[1] ^ x[0]
        x[0] = x[0] + ks[(i + 1) % 3]
        x[1] = x[1] + ks[(i + 2) % 3] + np.uint32(i + 1)
    return x


def _uniform_f32(seed, shape):
    size = int(np.prod(shape))
    k1 = np.uint32(np.uint64(seed) >> np.uint64(32))
    k2 = np.uint32(np.uint64(seed) & np.uint64(0xFFFFFFFF))
    hi = np.zeros(size, dtype=np.uint32)
    lo = np.arange(size, dtype=np.uint32)
    with np.errstate(over="ignore"):
        r = _threefry2x32(k1, k2, hi, lo)
    bits = r[0] ^ r[1]
    fb = (bits >> np.uint32(9)) | np.uint32(0x3F800000)
    return (fb.view(np.float32) - np.float32(1.0)).reshape(shape)


@functools.lru_cache(maxsize=1)
def _index_constants():
    noise = _uniform_f32(42, (_B, _NO))
    shuffle = np.argsort(noise, axis=-1, kind="stable").astype(np.int32)
    remain = shuffle[:, :_NUM_REMAIN]
    masked = shuffle[:, _NUM_REMAIN:]
    revert = np.argsort(shuffle, axis=-1, kind="stable").astype(np.int32)
    return remain, masked, revert


@functools.lru_cache(maxsize=1)
def _sc_gather_constants():
    remain, _, _ = _index_constants()
    b = np.repeat(np.arange(_B, dtype=np.int32), _NUM_REMAIN)
    j = np.tile(np.arange(_NUM_REMAIN, dtype=np.int32), _B)
    r = remain.reshape(-1)
    src_o = (r * _B + b).reshape(_NW * _NCH, _CH)
    src_pe = (3 + r).reshape(_NW * _NCH, _CH)
    dst_o = ((1 + j) * _B + b).reshape(_NW * _NCH, _CH)
    return src_o, src_pe, dst_o


def _tc_b_body(t_ref, i_ref, pe_ref, to_ref, io_ref):
    to_ref[...] = t_ref[...] + pe_ref[1:2, :]
    io_ref[...] = i_ref[...] + pe_ref[2:3, :]


def _sc_body(o_ref, pe_ref, gt_ref, src_o_ref, src_pe_ref, dst_o_ref,
             out_ref,
             idx_o, idx_pe, idx_d, rows, perows, gtbuf, gtrow, pe0row,
             gsem, ssem):
    w = lax.axis_index("s") * 2 + lax.axis_index("c")
    gathers = []
    for c in range(_NCH):
        pltpu.sync_copy(src_o_ref.at[w * _NCH + c], idx_o.at[c])
        pltpu.sync_copy(src_pe_ref.at[w * _NCH + c], idx_pe.at[c])
        gathers.append(pltpu.async_copy(
            o_ref.at[idx_o.at[c]], rows.at[c], gsem))
        gathers.append(pltpu.async_copy(
            pe_ref.at[idx_pe.at[c]], perows.at[c], gsem))
    pltpu.sync_copy(gt_ref, gtrow)
    pltpu.sync_copy(pe_ref.at[pl.ds(0, 1)], pe0row)

    def fill_gt(i, _):
        for v in range(8):
            gtbuf[i, pl.ds(v * 16, 16)] = (
                gtrow[0, pl.ds(v * 16, 16)] + pe0row[0, pl.ds(v * 16, 16)])
        return 0

    lax.fori_loop(0, _SW, fill_gt, 0)
    scatters = [pltpu.async_copy(gtbuf, out_ref.at[pl.ds(w * _SW, _SW)], ssem)]
    for g in gathers:
        g.wait()
    for c in range(_NCH):
        def add_pe(k, _, c=c):
            for v in range(8):
                rows[c, k, pl.ds(v * 16, 16)] = (
                    rows[c, k, pl.ds(v * 16, 16)]
                    + perows[c, k, pl.ds(v * 16, 16)])
            return 0

        lax.fori_loop(0, _CH, add_pe, 0)
        pltpu.sync_copy(dst_o_ref.at[w * _NCH + c], idx_d.at[c])
        scatters.append(pltpu.async_copy(
            rows.at[c], out_ref.at[idx_d.at[c]], ssem))
    for s in scatters:
        s.wait()


@jax.jit
def kernel(temporal_x, img_x, others_x, pos_emb, global_token):
    remain, masked, revert = _index_constants()
    src_o, src_pe, dst_o = _sc_gather_constants()

    o2 = others_x.transpose(1, 0, 2).reshape(_NO * _B, _D)
    i2 = img_x.transpose(1, 0, 2).reshape(_P * _B, _D)
    t2 = temporal_x.reshape(_B * _T, _D)

    mesh = plsc.VectorSubcoreMesh(core_axis_name="c", subcore_axis_name="s")
    or_flat = pl.kernel(
        _sc_body,
        out_type=jax.ShapeDtypeStruct(((1 + _NUM_REMAIN) * _B, _D),
                                      jnp.float32),
        mesh=mesh,
        scratch_types=[
            pltpu.VMEM((_NCH, _CH), jnp.int32),
            pltpu.VMEM((_NCH, _CH), jnp.int32),
            pltpu.VMEM((_NCH, _CH), jnp.int32),
            pltpu.VMEM((_NCH, _CH, _D), jnp.float32),
            pltpu.VMEM((_NCH, _CH, _D), jnp.float32),
            pltpu.VMEM((_SW, _D), jnp.float32),
            pltpu.VMEM((1, _D), jnp.float32),
            pltpu.VMEM((1, _D), jnp.float32),
            pltpu.SemaphoreType.DMA,
            pltpu.SemaphoreType.DMA,
        ],
    )(
        o2,
        pos_emb,
        global_token,
        jnp.asarray(src_o),
        jnp.asarray(src_pe),
        jnp.asarray(dst_o),
    )
    or_out = or_flat.reshape(1 + _NUM_REMAIN, _B, _D).transpose(1, 0, 2)

    t_out2, i_out2 = pl.pallas_call(
        _tc_b_body,
        grid=(_GB,),
        in_specs=[
            pl.BlockSpec((_B * _T // _GB, _D), lambda i: (i, 0)),
            pl.BlockSpec((_P * _B // _GB, _D), lambda i: (i, 0)),
            pl.BlockSpec((pos_emb.shape[0], _D), lambda i: (0, 0)),
        ],
        out_specs=[
            pl.BlockSpec((_B * _T // _GB, _D), lambda i: (i, 0)),
            pl.BlockSpec((_P * _B // _GB, _D), lambda i: (i, 0)),
        ],
        out_shape=[
            jax.ShapeDtypeStruct((_B * _T, _D), jnp.float32),
            jax.ShapeDtypeStruct((_P * _B, _D), jnp.float32),
        ],
    )(t2, i2, pos_emb)
    t_out = t_out2.reshape(_B, _T, _D)
    i_out = i_out2.reshape(_P, _B, _D).transpose(1, 0, 2)

    return (t_out, i_out, or_out,
            jnp.asarray(remain), jnp.asarray(masked), jnp.asarray(revert))
